# baseline (speedup 1.0000x reference)
import jax
import jax.numpy as jnp
from jax.experimental import pallas as pl


def _body(z_ref, mask_ref, out_ref):
    out_ref[...] = z_ref[...] * mask_ref[...]


def kernel(z, mask):
    B, S, D = z.shape
    rows = B * S
    z2 = z.reshape(rows, D)
    BR = 512
    grid = (rows // BR,)
    out = pl.pallas_call(
        _body,
        grid=grid,
        in_specs=[
            pl.BlockSpec((BR, D), lambda i: (i, 0)),
            pl.BlockSpec((1, D), lambda i: (0, 0)),
        ],
        out_specs=pl.BlockSpec((BR, D), lambda i: (i, 0)),
        out_shape=jax.ShapeDtypeStruct((rows, D), z.dtype),
    )(z2, mask.reshape(1, D))
    return out.reshape(B, S, D)
